# baseline (speedup 1.0000x reference)
import functools

import jax
import jax.numpy as jnp
from jax import lax
from jax.experimental import pallas as pl
from jax.experimental.pallas import tpu as pltpu

ATOM_VOCAB = 128
HYBRID_VOCAB = 8
ATOM_DIM = 32
HYBRID_DIM = 16
HIDDEN = 64


def _dg(a, b, dims):
    return lax.dot_general(a, b, (dims, ((), ())),
                           preferred_element_type=jnp.float32)


def _fused_body(aT_ref, hT_ref, xT_ref, at_ref, ht_ref, wc_ref, bc_ref,
                wf_ref, bfc_ref, wo_ref, boc_ref, o_ref,
                ta_s, th_s, w1t_s, b1c_s, wob_s, *, bt, na):
    @pl.when(jnp.logical_and(pl.program_id(0) == 0, pl.program_id(1) == 0))
    def _():
        wf_a = wf_ref[0:ATOM_DIM, :]
        wf_h = wf_ref[ATOM_DIM:ATOM_DIM + HYBRID_DIM, :]
        wf_c = wf_ref[ATOM_DIM + HYBRID_DIM:, :]
        ta_s[...] = _dg(wf_a, at_ref[...], ((0,), (1,))).astype(jnp.bfloat16)
        th_s[...] = _dg(wf_h, ht_ref[...], ((0,), (1,))).astype(jnp.bfloat16)
        w1t_s[...] = _dg(wf_c, wc_ref[...], ((0,), (1,)))
        b1c_s[...] = _dg(wf_c, bc_ref[...], ((0,), (1,))) + bfc_ref[...]
        wob_s[...] = wo_ref[...].astype(jnp.bfloat16)

    ta = ta_s[...]
    th = th_s[...]
    w1t = w1t_s[...]
    b1c = b1c_s[...]
    wob = wob_s[...]
    boc = boc_ref[...]
    j = pl.program_id(1)
    for a in range(na):
        aid = aT_ref[pl.ds(j * na + a, 1), :]
        hid = hT_ref[pl.ds(j * na + a, 1), :]
        oh_a = (aid == lax.broadcasted_iota(jnp.int32, (ATOM_VOCAB, bt), 0)).astype(jnp.bfloat16)
        oh_h = (hid == lax.broadcasted_iota(jnp.int32, (HYBRID_VOCAB, bt), 0)).astype(jnp.bfloat16)
        h1t = (_dg(ta, oh_a, ((1,), (0,)))
               + _dg(th, oh_h, ((1,), (0,)))
               + _dg(w1t, xT_ref[a], ((1,), (0,)))
               + b1c)
        h = jax.nn.gelu(h1t.astype(jnp.bfloat16), approximate=True)
        o_ref[a] = _dg(wob, h, ((0,), (0,))) + boc


def kernel(atom_ids, hybrid_ids, node_continuous, atom_table, hybrid_table,
           Wc, bc, Wf, bf, Wo, bo):
    B, A = atom_ids.shape
    BT = 2048
    NJ = 10
    assert B % BT == 0 and A % NJ == 0
    nb = B // BT
    na = A // NJ
    cont_in = node_continuous.shape[-1]

    aT = atom_ids.T
    hT = hybrid_ids.T
    xT = jnp.transpose(node_continuous, (1, 2, 0))
    bc2 = bc.reshape(1, -1)
    bfc = bf.reshape(-1, 1)
    boc = bo.reshape(-1, 1)

    rep = lambda shape: pl.BlockSpec(shape, lambda i, j: (0,) * len(shape))
    outT = pl.pallas_call(
        functools.partial(_fused_body, bt=BT, na=na),
        grid=(nb, NJ),
        in_specs=[
            pl.BlockSpec((A, BT), lambda i, j: (0, i)),
            pl.BlockSpec((A, BT), lambda i, j: (0, i)),
            pl.BlockSpec((na, cont_in, BT), lambda i, j: (j, 0, i)),
            rep(atom_table.shape),
            rep(hybrid_table.shape),
            rep(Wc.shape),
            rep(bc2.shape),
            rep(Wf.shape),
            rep(bfc.shape),
            rep(Wo.shape),
            rep(boc.shape),
        ],
        out_specs=pl.BlockSpec((na, HIDDEN, BT), lambda i, j: (j, 0, i)),
        out_shape=jax.ShapeDtypeStruct((A, HIDDEN, B), jnp.float32),
        scratch_shapes=[
            pltpu.VMEM((HIDDEN, ATOM_VOCAB), jnp.bfloat16),
            pltpu.VMEM((HIDDEN, HYBRID_VOCAB), jnp.bfloat16),
            pltpu.VMEM((HIDDEN, cont_in), jnp.float32),
            pltpu.VMEM((HIDDEN, 1), jnp.float32),
            pltpu.VMEM((HIDDEN, HIDDEN), jnp.bfloat16),
        ],
    )(aT, hT, xT, atom_table, hybrid_table, Wc, bc2, Wf, bfc, Wo, boc)
    return jnp.transpose(outT, (2, 0, 1))


# speedup vs baseline: 1.0258x; 1.0258x over previous
import functools

import jax
import jax.numpy as jnp
from jax import lax
from jax.experimental import pallas as pl
from jax.experimental.pallas import tpu as pltpu

ATOM_VOCAB = 128
HYBRID_VOCAB = 8
ATOM_DIM = 32
HYBRID_DIM = 16
HIDDEN = 64


def _dg(a, b, dims):
    return lax.dot_general(a, b, (dims, ((), ())),
                           preferred_element_type=jnp.float32)


def _fused_body(aT_ref, hT_ref, xT_ref, at_ref, ht_ref, wc_ref, bc_ref,
                wf_ref, bfc_ref, wo_ref, boc_ref, o_ref,
                ta_s, th_s, w1t_s, b1c_s, wob_s, *, bt, na):
    @pl.when(jnp.logical_and(pl.program_id(0) == 0, pl.program_id(1) == 0))
    def _():
        wf_a = wf_ref[0:ATOM_DIM, :]
        wf_h = wf_ref[ATOM_DIM:ATOM_DIM + HYBRID_DIM, :]
        wf_c = wf_ref[ATOM_DIM + HYBRID_DIM:, :]
        ta_s[...] = _dg(wf_a, at_ref[...], ((0,), (1,))).astype(jnp.bfloat16)
        th_s[...] = _dg(wf_h, ht_ref[...], ((0,), (1,))).astype(jnp.bfloat16)
        w1t_s[...] = _dg(wf_c, wc_ref[...], ((0,), (1,)))
        b1c_s[...] = _dg(wf_c, bc_ref[...], ((0,), (1,))) + bfc_ref[...]
        wob_s[...] = wo_ref[...].astype(jnp.bfloat16)

    ta = ta_s[...]
    th = th_s[...]
    w1t = w1t_s[...]
    b1c = b1c_s[...]
    wob = wob_s[...]
    boc = boc_ref[...]
    j = pl.program_id(1)
    for a in range(na):
        aid = aT_ref[pl.ds(j * na + a, 1), :]
        hid = hT_ref[pl.ds(j * na + a, 1), :]
        oh_a = (aid == lax.broadcasted_iota(jnp.int32, (ATOM_VOCAB, bt), 0)).astype(jnp.bfloat16)
        oh_h = (hid == lax.broadcasted_iota(jnp.int32, (HYBRID_VOCAB, bt), 0)).astype(jnp.bfloat16)
        h1t = (_dg(ta, oh_a, ((1,), (0,)))
               + _dg(th, oh_h, ((1,), (0,)))
               + _dg(w1t, xT_ref[a], ((1,), (0,)))
               + b1c)
        h = jax.nn.gelu(h1t.astype(jnp.bfloat16), approximate=True)
        o_ref[a] = _dg(wob, h, ((0,), (0,))) + boc


def kernel(atom_ids, hybrid_ids, node_continuous, atom_table, hybrid_table,
           Wc, bc, Wf, bf, Wo, bo):
    B, A = atom_ids.shape
    BT = 2048
    NJ = 5
    assert B % BT == 0 and A % NJ == 0
    nb = B // BT
    na = A // NJ
    cont_in = node_continuous.shape[-1]

    aT = atom_ids.T
    hT = hybrid_ids.T
    xT = jnp.transpose(node_continuous, (1, 2, 0))
    bc2 = bc.reshape(1, -1)
    bfc = bf.reshape(-1, 1)
    boc = bo.reshape(-1, 1)

    rep = lambda shape: pl.BlockSpec(shape, lambda i, j: (0,) * len(shape))
    outT = pl.pallas_call(
        functools.partial(_fused_body, bt=BT, na=na),
        grid=(nb, NJ),
        in_specs=[
            pl.BlockSpec((A, BT), lambda i, j: (0, i)),
            pl.BlockSpec((A, BT), lambda i, j: (0, i)),
            pl.BlockSpec((na, cont_in, BT), lambda i, j: (j, 0, i)),
            rep(atom_table.shape),
            rep(hybrid_table.shape),
            rep(Wc.shape),
            rep(bc2.shape),
            rep(Wf.shape),
            rep(bfc.shape),
            rep(Wo.shape),
            rep(boc.shape),
        ],
        out_specs=pl.BlockSpec((na, HIDDEN, BT), lambda i, j: (j, 0, i)),
        out_shape=jax.ShapeDtypeStruct((A, HIDDEN, B), jnp.float32),
        scratch_shapes=[
            pltpu.VMEM((HIDDEN, ATOM_VOCAB), jnp.bfloat16),
            pltpu.VMEM((HIDDEN, HYBRID_VOCAB), jnp.bfloat16),
            pltpu.VMEM((HIDDEN, cont_in), jnp.float32),
            pltpu.VMEM((HIDDEN, 1), jnp.float32),
            pltpu.VMEM((HIDDEN, HIDDEN), jnp.bfloat16),
        ],
    )(aT, hT, xT, atom_table, hybrid_table, Wc, bc2, Wf, bfc, Wo, boc)
    return jnp.transpose(outT, (2, 0, 1))
